# baseline (speedup 1.0000x reference)
import jax
import jax.numpy as jnp
from jax import lax
from jax.experimental import pallas as pl
from jax.experimental.pallas import tpu as pltpu
from jax.experimental.pallas import tpu_sc as plsc

_NC = 2
_NS = 16
_L = 16

_B, _F, _H, _WD = 16, 512, 32, 32
_P = _H * _WD
_NW = _NC * _NS
_FPW = _F // _NW


def _body(w_hbm, out_hbm, wflat, chunk, sem):
    wid = lax.axis_index("s") * _NC + lax.axis_index("c")
    fbase = wid * _FPW

    pltpu.sync_copy(w_hbm, wflat)

    lane = lax.iota(jnp.int32, _L)

    def step(pc, carry):
        p = pc * _L + lane
        i = p >> 5
        j = p & (_WD - 1)
        z = jnp.maximum(jnp.abs(_WD // 2 - j) + jnp.abs(_H // 2 - i) - 1, 0)
        zoff = z * _F + fbase
        p0 = pc * _L
        for ff in range(_FPW):
            v = plsc.load_gather(wflat, [zoff + ff])
            chunk[ff, pl.ds(p0, _L)] = v
        return carry

    lax.fori_loop(0, _P // _L, step, 0)

    copies = [
        pltpu.async_copy(chunk, out_hbm.at[b, pl.ds(fbase, _FPW), :], sem)
        for b in range(_B)
    ]
    for cp in copies:
        cp.wait()


def kernel(x, W):
    del x
    mesh = plsc.VectorSubcoreMesh(core_axis_name="c", subcore_axis_name="s")
    f = pl.kernel(
        _body,
        mesh=mesh,
        compiler_params=pltpu.CompilerParams(needs_layout_passes=False),
        out_type=jax.ShapeDtypeStruct((_B, _F, _P), jnp.float32),
        scratch_types=[
            pltpu.VMEM((_H * _F,), jnp.float32),
            pltpu.VMEM((_FPW, _P), jnp.float32),
            pltpu.SemaphoreType.DMA,
        ],
    )
    out = f(W.reshape(-1))
    return out.reshape(_B, _F, _H, _WD)
